# baseline (speedup 1.0000x reference)
import functools

import jax
import jax.numpy as jnp
from jax import lax
from jax.experimental import pallas as pl
from jax.experimental.pallas import tpu as pltpu
from jax.experimental.pallas import tpu_sc as plsc

_EMB = 16
_BATCH = 16384
_REGS = 1e-5
_NC = 2
_NS = 16
_NW = _NC * _NS
_BPW = _BATCH // _NW
_CHUNK = 128
_NCH = _BPW // _CHUNK
_GROUPS = _BPW // 16


def _sc_partials(table, u_idx, p_idx, n_idx):
    mesh = plsc.VectorSubcoreMesh(core_axis_name="c", subcore_axis_name="s")

    @functools.partial(
        pl.kernel,
        mesh=mesh,
        compiler_params=pltpu.CompilerParams(
            needs_layout_passes=False, use_tc_tiling_on_sc=False),
        out_type=jax.ShapeDtypeStruct((_NW, 16), jnp.float32),
        scratch_types=[
            pltpu.VMEM((_NCH, _CHUNK), jnp.int32),
            pltpu.VMEM((_NCH, _CHUNK), jnp.int32),
            pltpu.VMEM((_NCH, _CHUNK), jnp.int32),
            pltpu.VMEM((_BPW, _EMB), jnp.float32),
            pltpu.VMEM((_BPW, _EMB), jnp.float32),
            pltpu.VMEM((_BPW, _EMB), jnp.float32),
            pltpu.VMEM((16,), jnp.float32),
            pltpu.SemaphoreType.DMA,
        ],
    )
    def body(tab, uix, pix, nix, out, ui, pi, ni, ue, pe, ne, obuf, sem):
        wid = lax.axis_index("s") * _NC + lax.axis_index("c")
        pltpu.sync_copy(uix.at[wid], ui)
        pltpu.sync_copy(pix.at[wid], pi)
        pltpu.sync_copy(nix.at[wid], ni)
        copies = []
        for c in range(_NCH):
            sl = pl.ds(c * _CHUNK, _CHUNK)
            copies.append(pltpu.async_copy(tab.at[ui.at[c]], ue.at[sl], sem))
            copies.append(pltpu.async_copy(tab.at[pi.at[c]], pe.at[sl], sem))
            copies.append(pltpu.async_copy(tab.at[ni.at[c]], ne.at[sl], sem))
        for cp in copies:
            cp.wait()

        lanes = lax.iota(jnp.int32, 16)
        zeros = jnp.zeros((16,), jnp.float32)

        def group(g, carry):
            bpr_acc, sq_acc = carry
            rows = g * 16 + lanes
            dacc = zeros
            for j in range(_EMB):
                f = (lanes + j) & 15
                u = plsc.load_gather(ue, [rows, f])
                p = plsc.load_gather(pe, [rows, f])
                n = plsc.load_gather(ne, [rows, f])
                dacc = dacc + u * (p - n)
                sq_acc = sq_acc + (u * u + (p * p + n * n))
            a = -dacc
            t = jnp.exp(-jnp.abs(a))
            y = 1.0 + t
            z = 0.7 * t
            for _ in range(3):
                z = z - 1.0 + y * jnp.exp(-z)
            bpr_acc = bpr_acc + (jnp.maximum(a, 0.0) + z)
            return bpr_acc, sq_acc

        bpr_acc, sq_acc = lax.fori_loop(0, _GROUPS, group, (zeros, zeros))
        bpr_tot = jnp.sum(bpr_acc)
        sq_tot = jnp.sum(sq_acc)
        vals = jnp.where(lanes == 0, bpr_tot,
                         jnp.where(lanes == 1, sq_tot, 0.0))
        obuf[...] = vals
        pltpu.sync_copy(obuf, out.at[wid])

    return body(table, u_idx, p_idx, n_idx)


def kernel(all_embed, user, pos_item, neg_item):
    u3 = user.astype(jnp.int32).reshape(_NW, _NCH, _CHUNK)
    p3 = pos_item.astype(jnp.int32).reshape(_NW, _NCH, _CHUNK)
    n3 = neg_item.astype(jnp.int32).reshape(_NW, _NCH, _CHUNK)
    partials = _sc_partials(all_embed, u3, p3, n3)
    bpr_loss = jnp.sum(partials[:, 0]) / _BATCH
    reg_loss = _REGS * 0.5 * jnp.sum(partials[:, 1])
    return (bpr_loss, reg_loss)


# speedup vs baseline: 14.3945x; 14.3945x over previous
import functools

import jax
import jax.numpy as jnp
from jax import lax
from jax.experimental import pallas as pl
from jax.experimental.pallas import tpu as pltpu
from jax.experimental.pallas import tpu_sc as plsc

_EMB = 16
_BATCH = 16384
_REGS = 1e-5
_NC = 2
_NS = 16
_NW = _NC * _NS
_BPW = _BATCH // _NW
_NCH = _BPW // 128
_GROUPS = _BPW // 16
_NDMA = _BPW * _EMB // 128

_TILES = 2_000_000 // 128
_FBLOCK = _TILES * 1024


def _sc_partials(tab_flat, u_idx, p_idx, n_idx):
    mesh = plsc.VectorSubcoreMesh(core_axis_name="c", subcore_axis_name="s")

    @functools.partial(
        pl.kernel,
        mesh=mesh,
        compiler_params=pltpu.CompilerParams(
            needs_layout_passes=False, use_tc_tiling_on_sc=False),
        out_type=jax.ShapeDtypeStruct((_NW, 16), jnp.float32),
        scratch_types=[
            pltpu.VMEM((_NCH, 128), jnp.int32),
            pltpu.VMEM((_NCH, 128), jnp.int32),
            pltpu.VMEM((_NCH, 128), jnp.int32),
            pltpu.VMEM((_NDMA, 128), jnp.int32),
            pltpu.VMEM((_NDMA, 128), jnp.int32),
            pltpu.VMEM((_NDMA, 128), jnp.int32),
            pltpu.VMEM((_NDMA, 128), jnp.float32),
            pltpu.VMEM((_NDMA, 128), jnp.float32),
            pltpu.VMEM((_NDMA, 128), jnp.float32),
            pltpu.VMEM((16,), jnp.float32),
            pltpu.SemaphoreType.DMA,
        ],
    )
    def body(tab, uix, pix, nix, out, ui, pi, ni, uf, pf, nf,
             ue, pe, ne, obuf, sem):
        wid = lax.axis_index("s") * _NC + lax.axis_index("c")
        pltpu.sync_copy(uix.at[wid], ui)
        pltpu.sync_copy(pix.at[wid], pi)
        pltpu.sync_copy(nix.at[wid], ni)

        def build(g, _):
            for sidx, fidx in ((ui, uf), (pi, pf), (ni, nf)):
                rv = sidx[g // 8, pl.ds((g % 8) * 16, 16)]
                base = ((rv >> 7) << 10) + (rv & 127)
                for j in range(_EMB):
                    off = (j & 7) * 128 + (j >> 3) * _FBLOCK
                    fidx[2 * g + (j >> 3), pl.ds((j & 7) * 16, 16)] = base + off
            return 0

        for g in range(_GROUPS):
            build(g, 0)

        def fire(cr, _):
            pltpu.async_copy(tab.at[uf.at[cr]], ue.at[cr], sem)
            pltpu.async_copy(tab.at[pf.at[cr]], pe.at[cr], sem)
            pltpu.async_copy(tab.at[nf.at[cr]], ne.at[cr], sem)
            return 0

        lax.fori_loop(0, _NDMA, fire, 0)

        def drain(cr, _):
            pltpu.make_async_copy(tab.at[uf.at[cr]], ue.at[cr], sem).wait()
            pltpu.make_async_copy(tab.at[pf.at[cr]], pe.at[cr], sem).wait()
            pltpu.make_async_copy(tab.at[nf.at[cr]], ne.at[cr], sem).wait()
            return 0

        lax.fori_loop(0, _NDMA, drain, 0)

        lanes = lax.iota(jnp.int32, 16)
        zeros = jnp.zeros((16,), jnp.float32)

        def group(g, carry):
            bpr_acc, sq_acc = carry
            dacc = zeros
            for j in range(_EMB):
                row = 2 * g + (j >> 3)
                sl = pl.ds((j & 7) * 16, 16)
                u = ue[row, sl]
                p = pe[row, sl]
                n = ne[row, sl]
                dacc = dacc + u * (p - n)
                sq_acc = sq_acc + (u * u + (p * p + n * n))
            a = -dacc
            t = jnp.exp(-jnp.abs(a))
            y = 1.0 + t
            z = 0.7 * t
            for _ in range(3):
                z = z - 1.0 + y * jnp.exp(-z)
            bpr_acc = bpr_acc + (jnp.maximum(a, 0.0) + z)
            return bpr_acc, sq_acc

        bpr_acc, sq_acc = lax.fori_loop(0, _GROUPS, group, (zeros, zeros))
        bpr_tot = jnp.sum(bpr_acc)
        sq_tot = jnp.sum(sq_acc)
        vals = jnp.where(lanes == 0, bpr_tot,
                         jnp.where(lanes == 1, sq_tot, 0.0))
        obuf[...] = vals
        pltpu.sync_copy(obuf, out.at[wid])

    return body(tab_flat, u_idx, p_idx, n_idx)


def kernel(all_embed, user, pos_item, neg_item):
    tab_flat = (
        all_embed.T.reshape(2, 8, _TILES, 128)
        .transpose(0, 2, 1, 3)
        .reshape(2 * _FBLOCK)
    )
    u3 = user.astype(jnp.int32).reshape(_NW, _NCH, 128)
    p3 = pos_item.astype(jnp.int32).reshape(_NW, _NCH, 128)
    n3 = neg_item.astype(jnp.int32).reshape(_NW, _NCH, 128)
    partials = _sc_partials(tab_flat, u3, p3, n3)
    bpr_loss = jnp.sum(partials[:, 0]) / _BATCH
    reg_loss = _REGS * 0.5 * jnp.sum(partials[:, 1])
    return (bpr_loss, reg_loss)


# speedup vs baseline: 15.1482x; 1.0524x over previous
import functools

import jax
import jax.numpy as jnp
from jax import lax
from jax.experimental import pallas as pl
from jax.experimental.pallas import tpu as pltpu
from jax.experimental.pallas import tpu_sc as plsc

_EMB = 16
_BATCH = 16384
_REGS = 1e-5
_NC = 2
_NS = 16
_NW = _NC * _NS
_BPW = _BATCH // _NW
_GROUPS = _BPW // 16
_ELEMS = _BPW * _EMB

_TILES = 2_000_000 // 128
_FBLOCK = _TILES * 1024


def _sc_partials(tab_flat, u_idx, p_idx, n_idx):
    mesh = plsc.VectorSubcoreMesh(core_axis_name="c", subcore_axis_name="s")

    @functools.partial(
        pl.kernel,
        mesh=mesh,
        compiler_params=pltpu.CompilerParams(
            needs_layout_passes=False, use_tc_tiling_on_sc=False),
        out_type=jax.ShapeDtypeStruct((_NW, 16), jnp.float32),
        scratch_types=[
            pltpu.VMEM((_BPW,), jnp.int32),
            pltpu.VMEM((_BPW,), jnp.int32),
            pltpu.VMEM((_BPW,), jnp.int32),
            pltpu.VMEM((_ELEMS,), jnp.int32),
            pltpu.VMEM((_ELEMS,), jnp.int32),
            pltpu.VMEM((_ELEMS,), jnp.int32),
            pltpu.VMEM((_ELEMS,), jnp.float32),
            pltpu.VMEM((_ELEMS,), jnp.float32),
            pltpu.VMEM((_ELEMS,), jnp.float32),
            pltpu.VMEM((16,), jnp.float32),
            pltpu.SemaphoreType.DMA,
        ],
    )
    def body(tab, uix, pix, nix, out, ui, pi, ni, uf, pf, nf,
             ue, pe, ne, obuf, sem):
        wid = lax.axis_index("s") * _NC + lax.axis_index("c")
        pltpu.sync_copy(uix.at[wid], ui)
        pltpu.sync_copy(pix.at[wid], pi)
        pltpu.sync_copy(nix.at[wid], ni)

        def build(g, _):
            for sidx, fidx in ((ui, uf), (pi, pf), (ni, nf)):
                rv = sidx[pl.ds(g * 16, 16)]
                base = ((rv >> 7) << 10) + (rv & 127)
                for j in range(_EMB):
                    off = (j & 7) * 128 + (j >> 3) * _FBLOCK
                    fidx[pl.ds(g * 256 + j * 16, 16)] = base + off
            return 0

        lax.fori_loop(0, _GROUPS, build, 0)

        pltpu.async_copy(tab.at[uf], ue, sem)
        pltpu.async_copy(tab.at[pf], pe, sem)
        pltpu.async_copy(tab.at[nf], ne, sem)
        pltpu.make_async_copy(tab.at[uf], ue, sem).wait()
        pltpu.make_async_copy(tab.at[pf], pe, sem).wait()
        pltpu.make_async_copy(tab.at[nf], ne, sem).wait()

        lanes = lax.iota(jnp.int32, 16)
        zeros = jnp.zeros((16,), jnp.float32)

        def group(g, carry):
            bpr_acc, sq_acc = carry
            dacc = zeros
            for j in range(_EMB):
                sl = pl.ds(g * 256 + j * 16, 16)
                u = ue[sl]
                p = pe[sl]
                n = ne[sl]
                dacc = dacc + u * (p - n)
                sq_acc = sq_acc + (u * u + (p * p + n * n))
            a = -dacc
            t = jnp.exp(-jnp.abs(a))
            y = 1.0 + t
            z = 0.7 * t
            for _ in range(3):
                z = z - 1.0 + y * jnp.exp(-z)
            bpr_acc = bpr_acc + (jnp.maximum(a, 0.0) + z)
            return bpr_acc, sq_acc

        bpr_acc, sq_acc = lax.fori_loop(0, _GROUPS, group, (zeros, zeros))
        bpr_tot = jnp.sum(bpr_acc)
        sq_tot = jnp.sum(sq_acc)
        vals = jnp.where(lanes == 0, bpr_tot,
                         jnp.where(lanes == 1, sq_tot, 0.0))
        obuf[...] = vals
        pltpu.sync_copy(obuf, out.at[wid])

    return body(tab_flat, u_idx, p_idx, n_idx)


def kernel(all_embed, user, pos_item, neg_item):
    tab_flat = (
        all_embed.T.reshape(2, 8, _TILES, 128)
        .transpose(0, 2, 1, 3)
        .reshape(2 * _FBLOCK)
    )
    u2 = user.astype(jnp.int32).reshape(_NW, _BPW)
    p2 = pos_item.astype(jnp.int32).reshape(_NW, _BPW)
    n2 = neg_item.astype(jnp.int32).reshape(_NW, _BPW)
    partials = _sc_partials(tab_flat, u2, p2, n2)
    bpr_loss = jnp.sum(partials[:, 0]) / _BATCH
    reg_loss = _REGS * 0.5 * jnp.sum(partials[:, 1])
    return (bpr_loss, reg_loss)
